# baseline (speedup 1.0000x reference)
import functools
import math

import jax
import jax.numpy as jnp
from jax import lax
from jax.experimental import pallas as pl
from jax.experimental.pallas import tpu as pltpu
from jax.experimental.pallas import tpu_sc as plsc

_EPS = 1.0
_ALPHA = 0.1
_DELTA = 0.1

_NC = 2
_NS = 16
_CHUNK = 128
_ROWBLK = 512

_mesh = plsc.VectorSubcoreMesh(core_axis_name="c", subcore_axis_name="s",
                               num_cores=_NC, num_subcores=_NS)


def _sc_degree(idxpack, ones_blk, zeros16, npad, epad):
    chunks = epad // (_NC * _NS * _CHUNK)
    stripe = npad // _NS
    zchunks = stripe // _CHUNK

    @functools.partial(
        pl.kernel,
        out_type=jax.ShapeDtypeStruct((_NC, npad, 16), jnp.float32),
        mesh=_mesh,
        scratch_types=[pltpu.VMEM((2, 2, _CHUNK), jnp.int32),
                       pltpu.VMEM((_CHUNK, 16), jnp.float32),
                       pltpu.VMEM_SHARED((npad, 16), jnp.float32),
                       pltpu.SemaphoreType.DMA,
                       pltpu.SemaphoreType.DMA],
    )
    def deg_kernel(idx_hbm, ones_hbm, zeros_hbm, out_hbm, idx_v, ones_v, acc,
                   isem0, isem1):
        c = lax.axis_index("c")
        s = lax.axis_index("s")
        cid0 = (c * _NS + s) * chunks
        pltpu.sync_copy(ones_hbm, ones_v)

        @pl.loop(0, zchunks)
        def _(k):
            pltpu.sync_copy(zeros_hbm, acc.at[pl.ds(s * stripe + k * _CHUNK, _CHUNK), :])

        plsc.subcore_barrier()

        def scat(slot):
            pltpu.sync_copy(ones_v, acc.at[idx_v.at[slot, 0]], add=True)

        pltpu.async_copy(idx_hbm.at[cid0], idx_v.at[0], isem0).wait()

        @pl.loop(0, chunks // 2 - 1)
        def _(k):
            pltpu.async_copy(idx_hbm.at[cid0 + 2 * k + 1], idx_v.at[1], isem1)
            scat(0)
            pltpu.make_async_copy(idx_hbm.at[cid0], idx_v.at[1], isem1).wait()
            pltpu.async_copy(idx_hbm.at[cid0 + 2 * k + 2], idx_v.at[0], isem0)
            scat(1)
            pltpu.make_async_copy(idx_hbm.at[cid0], idx_v.at[0], isem0).wait()

        pltpu.async_copy(idx_hbm.at[cid0 + chunks - 1], idx_v.at[1], isem1)
        scat(0)
        pltpu.make_async_copy(idx_hbm.at[cid0], idx_v.at[1], isem1).wait()
        scat(1)

        plsc.subcore_barrier()

        @pl.loop(0, zchunks)
        def _(k):
            r0 = s * stripe + k * _CHUNK
            pltpu.sync_copy(acc.at[pl.ds(r0, _CHUNK), :],
                            out_hbm.at[c, pl.ds(r0, _CHUNK), :])

    return deg_kernel(idxpack, ones_blk, zeros16)


def _sc_aggregate(table, idxpack, zeros128, npad, epad):
    chunks = epad // (_NC * _NS * _CHUNK)
    stripe = npad // _NS
    zchunks = stripe // _CHUNK

    @functools.partial(
        pl.kernel,
        out_type=jax.ShapeDtypeStruct((_NC, npad, 128), jnp.float32),
        mesh=_mesh,
        scratch_types=[pltpu.VMEM((2, 2, _CHUNK), jnp.int32),
                       pltpu.VMEM((2, _CHUNK, 128), jnp.float32),
                       pltpu.VMEM_SHARED((npad, 128), jnp.float32),
                       pltpu.SemaphoreType.DMA,
                       pltpu.SemaphoreType.DMA],
    )
    def agg_kernel(table_hbm, idx_hbm, zeros_hbm, out_hbm,
                   idx_v, buf, acc, gsem0, gsem1):
        c = lax.axis_index("c")
        s = lax.axis_index("s")
        cid0 = (c * _NS + s) * chunks

        @pl.loop(0, zchunks)
        def _(k):
            pltpu.sync_copy(zeros_hbm, acc.at[pl.ds(s * stripe + k * _CHUNK, _CHUNK), :])

        plsc.subcore_barrier()

        def load_and_gather(cid, slot, sem):
            pltpu.sync_copy(idx_hbm.at[cid], idx_v.at[slot])
            pltpu.async_copy(table_hbm.at[idx_v.at[slot, 0]], buf.at[slot], sem)

        def wait_gather(slot, sem):
            pltpu.make_async_copy(zeros_hbm, buf.at[slot], sem).wait()

        def scat(slot):
            pltpu.sync_copy(buf.at[slot], acc.at[idx_v.at[slot, 1]], add=True)

        load_and_gather(cid0, 0, gsem0)

        @pl.loop(0, chunks // 2 - 1)
        def _(k):
            load_and_gather(cid0 + 2 * k + 1, 1, gsem1)
            wait_gather(0, gsem0)
            scat(0)
            load_and_gather(cid0 + 2 * k + 2, 0, gsem0)
            wait_gather(1, gsem1)
            scat(1)

        load_and_gather(cid0 + chunks - 1, 1, gsem1)
        wait_gather(0, gsem0)
        scat(0)
        wait_gather(1, gsem1)
        scat(1)

        plsc.subcore_barrier()

        @pl.loop(0, zchunks)
        def _(k):
            r0 = s * stripe + k * _CHUNK
            pltpu.sync_copy(acc.at[pl.ds(r0, _CHUNK), :],
                            out_hbm.at[c, pl.ds(r0, _CHUNK), :])

    return agg_kernel(table, idxpack, zeros128)


def _tc_build_z(d0, d1, x, pf, n, npad, a, c_t):
    grid = npad // _ROWBLK

    def body(d0_ref, d1_ref, x_ref, pf_ref, z_ref):
        i = pl.program_id(0)
        deg = d0_ref[:, 0:1] + d1_ref[:, 0:1] + 1.0
        dinv = lax.rsqrt(deg)
        pf = pf_ref[...]
        sv = jnp.where(pf > 0, a, 1.0)
        tv = jnp.where(pf > 0, c_t, 0.0)
        z = dinv * (sv * x_ref[...] + tv)
        rows = i * _ROWBLK + lax.broadcasted_iota(jnp.int32, (_ROWBLK, 128), 0)
        z_ref[...] = jnp.where(rows < n, z, 0.0)

    return pl.pallas_call(
        body,
        grid=(grid,),
        in_specs=[pl.BlockSpec((_ROWBLK, 16), lambda i: (i, 0)),
                  pl.BlockSpec((_ROWBLK, 16), lambda i: (i, 0)),
                  pl.BlockSpec((_ROWBLK, 128), lambda i: (i, 0)),
                  pl.BlockSpec((_ROWBLK, 1), lambda i: (i, 0))],
        out_specs=pl.BlockSpec((_ROWBLK, 128), lambda i: (i, 0)),
        out_shape=jax.ShapeDtypeStruct((npad, 128), jnp.float32),
    )(d0, d1, x, pf)


def _tc_mid(d0, d1, a0, a1, z, w1, b1, w2, n, npad):
    grid = npad // _ROWBLK

    def body(d0_ref, d1_ref, a0_ref, a1_ref, z_ref, w1_ref, b1_ref, w2_ref,
             z2_ref):
        i = pl.program_id(0)
        deg = d0_ref[:, 0:1] + d1_ref[:, 0:1] + 1.0
        dinv = lax.rsqrt(deg)
        h1 = dinv * (a0_ref[...] + a1_ref[...] + z_ref[...])
        h = lax.dot_general(h1, w1_ref[...], (((1,), (1,)), ((), ())),
                            preferred_element_type=jnp.float32)
        h = jnp.maximum(h + b1_ref[...], 0.0)
        h2 = lax.dot_general(h, w2_ref[...], (((1,), (1,)), ((), ())),
                             preferred_element_type=jnp.float32)
        rows = i * _ROWBLK + lax.broadcasted_iota(jnp.int32, (_ROWBLK, 128), 0)
        z2_ref[...] = jnp.where(rows < n, dinv * h2, 0.0)

    return pl.pallas_call(
        body,
        grid=(grid,),
        in_specs=[pl.BlockSpec((_ROWBLK, 16), lambda i: (i, 0)),
                  pl.BlockSpec((_ROWBLK, 16), lambda i: (i, 0)),
                  pl.BlockSpec((_ROWBLK, 128), lambda i: (i, 0)),
                  pl.BlockSpec((_ROWBLK, 128), lambda i: (i, 0)),
                  pl.BlockSpec((_ROWBLK, 128), lambda i: (i, 0)),
                  pl.BlockSpec((128, 128), lambda i: (0, 0)),
                  pl.BlockSpec((1, 128), lambda i: (0, 0)),
                  pl.BlockSpec((128, 128), lambda i: (0, 0))],
        out_specs=pl.BlockSpec((_ROWBLK, 128), lambda i: (i, 0)),
        out_shape=jax.ShapeDtypeStruct((npad, 128), jnp.float32),
    )(d0, d1, a0, a1, z, w1, b1, w2)


def _tc_out(d0, d1, a0, a1, z2, b2, npad):
    grid = npad // _ROWBLK

    def body(d0_ref, d1_ref, a0_ref, a1_ref, z2_ref, b2_ref, o_ref):
        deg = d0_ref[:, 0:1] + d1_ref[:, 0:1] + 1.0
        dinv = lax.rsqrt(deg)
        o_ref[...] = dinv * (a0_ref[...] + a1_ref[...] + z2_ref[...]) + b2_ref[...]

    return pl.pallas_call(
        body,
        grid=(grid,),
        in_specs=[pl.BlockSpec((_ROWBLK, 16), lambda i: (i, 0)),
                  pl.BlockSpec((_ROWBLK, 16), lambda i: (i, 0)),
                  pl.BlockSpec((_ROWBLK, 128), lambda i: (i, 0)),
                  pl.BlockSpec((_ROWBLK, 128), lambda i: (i, 0)),
                  pl.BlockSpec((_ROWBLK, 128), lambda i: (i, 0)),
                  pl.BlockSpec((1, 128), lambda i: (0, 0))],
        out_specs=pl.BlockSpec((_ROWBLK, 128), lambda i: (i, 0)),
        out_shape=jax.ShapeDtypeStruct((npad, 128), jnp.float32),
    )(d0, d1, a0, a1, z2, b2)


def kernel(x, edge_index, priv_mask, W1, b1, W2, b2):
    n = x.shape[0]
    e = edge_index.shape[1]
    npad = -(-(n + 1) // (_NS * _CHUNK)) * (_NS * _CHUNK)
    epad = -(-e // (2 * _NC * _NS * _CHUNK)) * (2 * _NC * _NS * _CHUNK)

    expv = math.exp(_EPS)
    a = (expv + 1.0) * _DELTA / (expv - 1.0)
    c_t = -_DELTA / (expv - 1.0) + _ALPHA

    row = edge_index[0].astype(jnp.int32)
    col = edge_index[1].astype(jnp.int32)
    rowr = jnp.concatenate(
        [jnp.where(row == col, n, row), jnp.full((epad - e,), n, jnp.int32)])
    colp = jnp.concatenate([col, jnp.full((epad - e,), n, jnp.int32)])
    idxpack = jnp.stack(
        [rowr.reshape(-1, _CHUNK), colp.reshape(-1, _CHUNK)], axis=1)

    ones_blk = jnp.ones((_CHUNK, 16), jnp.float32)
    zeros16 = jnp.zeros((_CHUNK, 16), jnp.float32)
    zeros128 = jnp.zeros((_CHUNK, 128), jnp.float32)

    degp = _sc_degree(idxpack, ones_blk, zeros16, npad, epad)
    d0, d1 = degp[0], degp[1]

    pf = priv_mask.astype(jnp.float32)
    z = _tc_build_z(d0, d1, x, pf, n, npad, a, c_t)

    agg1 = _sc_aggregate(z, idxpack, zeros128, npad, epad)
    z2 = _tc_mid(d0, d1, agg1[0], agg1[1], z, W1, b1.reshape(1, 128),
                 W2, n, npad)

    agg2 = _sc_aggregate(z2, idxpack, zeros128, npad, epad)
    outp = _tc_out(d0, d1, agg2[0], agg2[1], z2, b2.reshape(1, 128), npad)
    return outp[:n]


# speedup vs baseline: 2.4491x; 2.4491x over previous
import functools
import math

import jax
import jax.numpy as jnp
from jax import lax
from jax.experimental import pallas as pl
from jax.experimental.pallas import tpu as pltpu
from jax.experimental.pallas import tpu_sc as plsc

_EPS = 1.0
_ALPHA = 0.1
_DELTA = 0.1

_NC = 2
_NS = 16
_CHUNK = 128
_ROWBLK = 512

_mesh = plsc.VectorSubcoreMesh(core_axis_name="c", subcore_axis_name="s",
                               num_cores=_NC, num_subcores=_NS)


def _sc_degree(idxpack, ones_blk, zeros16, npad, epad):
    chunks = epad // (_NC * _NS * _CHUNK)
    stripe = npad // _NS
    zchunks = stripe // _CHUNK

    @functools.partial(
        pl.kernel,
        out_type=jax.ShapeDtypeStruct((_NC, npad, 16), jnp.float32),
        mesh=_mesh,
        scratch_types=[pltpu.VMEM((2, 2, _CHUNK), jnp.int32),
                       pltpu.VMEM((_CHUNK, 16), jnp.float32),
                       pltpu.VMEM_SHARED((npad, 16), jnp.float32),
                       pltpu.SemaphoreType.DMA,
                       pltpu.SemaphoreType.DMA],
    )
    def deg_kernel(idx_hbm, ones_hbm, zeros_hbm, out_hbm, idx_v, ones_v, acc,
                   isem0, isem1):
        c = lax.axis_index("c")
        s = lax.axis_index("s")
        cid0 = (c * _NS + s) * chunks
        pltpu.sync_copy(ones_hbm, ones_v)

        @pl.loop(0, zchunks)
        def _(k):
            pltpu.sync_copy(zeros_hbm, acc.at[pl.ds(s * stripe + k * _CHUNK, _CHUNK), :])

        plsc.subcore_barrier()

        def scat(slot):
            pltpu.sync_copy(ones_v, acc.at[idx_v.at[slot, 0]], add=True)

        pltpu.async_copy(idx_hbm.at[cid0], idx_v.at[0], isem0).wait()

        @pl.loop(0, chunks // 2 - 1)
        def _(k):
            pltpu.async_copy(idx_hbm.at[cid0 + 2 * k + 1], idx_v.at[1], isem1)
            scat(0)
            pltpu.make_async_copy(idx_hbm.at[cid0], idx_v.at[1], isem1).wait()
            pltpu.async_copy(idx_hbm.at[cid0 + 2 * k + 2], idx_v.at[0], isem0)
            scat(1)
            pltpu.make_async_copy(idx_hbm.at[cid0], idx_v.at[0], isem0).wait()

        pltpu.async_copy(idx_hbm.at[cid0 + chunks - 1], idx_v.at[1], isem1)
        scat(0)
        pltpu.make_async_copy(idx_hbm.at[cid0], idx_v.at[1], isem1).wait()
        scat(1)

        plsc.subcore_barrier()

        @pl.loop(0, zchunks)
        def _(k):
            r0 = s * stripe + k * _CHUNK
            pltpu.sync_copy(acc.at[pl.ds(r0, _CHUNK), :],
                            out_hbm.at[c, pl.ds(r0, _CHUNK), :])

    return deg_kernel(idxpack, ones_blk, zeros16)


def _sc_aggregate(table, idxpack, zeros128, npad, epad):
    chunks = epad // (_NC * _NS * _CHUNK)
    stripe = npad // _NS
    zchunks = stripe // _CHUNK

    @functools.partial(
        pl.kernel,
        out_type=jax.ShapeDtypeStruct((_NC, npad, 128), jnp.float32),
        mesh=_mesh,
        scratch_types=[pltpu.VMEM((2, 2, _CHUNK), jnp.int32),
                       pltpu.VMEM((2, _CHUNK, 128), jnp.float32),
                       pltpu.VMEM_SHARED((npad, 128), jnp.float32),
                       pltpu.SemaphoreType.DMA,
                       pltpu.SemaphoreType.DMA],
    )
    def agg_kernel(table_hbm, idx_hbm, zeros_hbm, out_hbm,
                   idx_v, buf, acc, gsem0, gsem1):
        c = lax.axis_index("c")
        s = lax.axis_index("s")
        cid0 = (c * _NS + s) * chunks

        @pl.loop(0, zchunks)
        def _(k):
            pltpu.sync_copy(zeros_hbm, acc.at[pl.ds(s * stripe + k * _CHUNK, _CHUNK), :])

        plsc.subcore_barrier()

        def load_and_gather(cid, slot, sem):
            pltpu.sync_copy(idx_hbm.at[cid], idx_v.at[slot])
            pltpu.async_copy(table_hbm.at[idx_v.at[slot, 0]], buf.at[slot], sem)

        def wait_gather(slot, sem):
            pltpu.make_async_copy(zeros_hbm, buf.at[slot], sem).wait()

        def scat(slot):
            pltpu.sync_copy(buf.at[slot], acc.at[idx_v.at[slot, 1]], add=True)

        load_and_gather(cid0, 0, gsem0)

        @pl.loop(0, chunks // 2 - 1)
        def _(k):
            load_and_gather(cid0 + 2 * k + 1, 1, gsem1)
            wait_gather(0, gsem0)
            scat(0)
            load_and_gather(cid0 + 2 * k + 2, 0, gsem0)
            wait_gather(1, gsem1)
            scat(1)

        load_and_gather(cid0 + chunks - 1, 1, gsem1)
        wait_gather(0, gsem0)
        scat(0)
        wait_gather(1, gsem1)
        scat(1)

        plsc.subcore_barrier()

        @pl.loop(0, zchunks)
        def _(k):
            r0 = s * stripe + k * _CHUNK
            pltpu.sync_copy(acc.at[pl.ds(r0, _CHUNK), :],
                            out_hbm.at[c, pl.ds(r0, _CHUNK), :])

    return agg_kernel(table, idxpack, zeros128)


def _tc_build_z(d0, d1, x, pf, n, npad, a, c_t):
    grid = npad // _ROWBLK

    def body(d0_ref, d1_ref, x_ref, pf_ref, z_ref):
        i = pl.program_id(0)
        deg = d0_ref[:, 0:1] + d1_ref[:, 0:1] + 1.0
        dinv = lax.rsqrt(deg)
        pf = pf_ref[...]
        sv = jnp.where(pf > 0, a, 1.0)
        tv = jnp.where(pf > 0, c_t, 0.0)
        z = dinv * (sv * x_ref[...] + tv)
        rows = i * _ROWBLK + lax.broadcasted_iota(jnp.int32, (_ROWBLK, 128), 0)
        z_ref[...] = jnp.where(rows < n, z, 0.0)

    return pl.pallas_call(
        body,
        grid=(grid,),
        in_specs=[pl.BlockSpec((_ROWBLK, 16), lambda i: (i, 0)),
                  pl.BlockSpec((_ROWBLK, 16), lambda i: (i, 0)),
                  pl.BlockSpec((_ROWBLK, 128), lambda i: (i, 0)),
                  pl.BlockSpec((_ROWBLK, 1), lambda i: (i, 0))],
        out_specs=pl.BlockSpec((_ROWBLK, 128), lambda i: (i, 0)),
        out_shape=jax.ShapeDtypeStruct((npad, 128), jnp.float32),
    )(d0, d1, x, pf)


def _tc_mid(d0, d1, a0, a1, z, w1, b1, w2, n, npad):
    grid = npad // _ROWBLK

    def body(d0_ref, d1_ref, a0_ref, a1_ref, z_ref, w1_ref, b1_ref, w2_ref,
             z2_ref):
        i = pl.program_id(0)
        deg = d0_ref[:, 0:1] + d1_ref[:, 0:1] + 1.0
        dinv = lax.rsqrt(deg)
        h1 = dinv * (a0_ref[...] + a1_ref[...] + z_ref[...])
        h = lax.dot_general(h1, w1_ref[...], (((1,), (1,)), ((), ())),
                            preferred_element_type=jnp.float32)
        h = jnp.maximum(h + b1_ref[...], 0.0)
        h2 = lax.dot_general(h, w2_ref[...], (((1,), (1,)), ((), ())),
                             preferred_element_type=jnp.float32)
        rows = i * _ROWBLK + lax.broadcasted_iota(jnp.int32, (_ROWBLK, 128), 0)
        z2_ref[...] = jnp.where(rows < n, dinv * h2, 0.0)

    return pl.pallas_call(
        body,
        grid=(grid,),
        in_specs=[pl.BlockSpec((_ROWBLK, 16), lambda i: (i, 0)),
                  pl.BlockSpec((_ROWBLK, 16), lambda i: (i, 0)),
                  pl.BlockSpec((_ROWBLK, 128), lambda i: (i, 0)),
                  pl.BlockSpec((_ROWBLK, 128), lambda i: (i, 0)),
                  pl.BlockSpec((_ROWBLK, 128), lambda i: (i, 0)),
                  pl.BlockSpec((128, 128), lambda i: (0, 0)),
                  pl.BlockSpec((1, 128), lambda i: (0, 0)),
                  pl.BlockSpec((128, 128), lambda i: (0, 0))],
        out_specs=pl.BlockSpec((_ROWBLK, 128), lambda i: (i, 0)),
        out_shape=jax.ShapeDtypeStruct((npad, 128), jnp.float32),
    )(d0, d1, a0, a1, z, w1, b1, w2)


def _tc_out(d0, d1, a0, a1, z2, b2, npad):
    grid = npad // _ROWBLK

    def body(d0_ref, d1_ref, a0_ref, a1_ref, z2_ref, b2_ref, o_ref):
        deg = d0_ref[:, 0:1] + d1_ref[:, 0:1] + 1.0
        dinv = lax.rsqrt(deg)
        o_ref[...] = dinv * (a0_ref[...] + a1_ref[...] + z2_ref[...]) + b2_ref[...]

    return pl.pallas_call(
        body,
        grid=(grid,),
        in_specs=[pl.BlockSpec((_ROWBLK, 16), lambda i: (i, 0)),
                  pl.BlockSpec((_ROWBLK, 16), lambda i: (i, 0)),
                  pl.BlockSpec((_ROWBLK, 128), lambda i: (i, 0)),
                  pl.BlockSpec((_ROWBLK, 128), lambda i: (i, 0)),
                  pl.BlockSpec((_ROWBLK, 128), lambda i: (i, 0)),
                  pl.BlockSpec((1, 128), lambda i: (0, 0))],
        out_specs=pl.BlockSpec((_ROWBLK, 128), lambda i: (i, 0)),
        out_shape=jax.ShapeDtypeStruct((npad, 128), jnp.float32),
    )(d0, d1, a0, a1, z2, b2)


def kernel(x, edge_index, priv_mask, W1, b1, W2, b2):
    n = x.shape[0]
    e = edge_index.shape[1]
    npad = -(-(n + 1) // (_NS * _CHUNK)) * (_NS * _CHUNK)
    epad = -(-e // (2 * _NC * _NS * _CHUNK)) * (2 * _NC * _NS * _CHUNK)

    expv = math.exp(_EPS)
    a = (expv + 1.0) * _DELTA / (expv - 1.0)
    c_t = -_DELTA / (expv - 1.0) + _ALPHA

    row = edge_index[0].astype(jnp.int32)
    col = edge_index[1].astype(jnp.int32)
    pad_idx = n + (jnp.arange(epad - e, dtype=jnp.int32) % 128)
    rowr = jnp.concatenate([jnp.where(row == col, n, row), pad_idx])
    colp = jnp.concatenate([col, pad_idx])
    idxpack = jnp.stack(
        [rowr.reshape(-1, _CHUNK), colp.reshape(-1, _CHUNK)], axis=1)

    ones_blk = jnp.ones((_CHUNK, 16), jnp.float32)
    zeros16 = jnp.zeros((_CHUNK, 16), jnp.float32)
    zeros128 = jnp.zeros((_CHUNK, 128), jnp.float32)

    degp = _sc_degree(idxpack, ones_blk, zeros16, npad, epad)
    d0, d1 = degp[0], degp[1]

    pf = priv_mask.astype(jnp.float32)
    z = _tc_build_z(d0, d1, x, pf, n, npad, a, c_t)

    agg1 = _sc_aggregate(z, idxpack, zeros128, npad, epad)
    z2 = _tc_mid(d0, d1, agg1[0], agg1[1], z, W1, b1.reshape(1, 128),
                 W2, n, npad)

    agg2 = _sc_aggregate(z2, idxpack, zeros128, npad, epad)
    outp = _tc_out(d0, d1, agg2[0], agg2[1], z2, b2.reshape(1, 128), npad)
    return outp[:n]
